# baseline (speedup 1.0000x reference)
import jax
import jax.numpy as jnp
from jax import lax
from jax.experimental import pallas as pl
from jax.experimental.pallas import tpu as pltpu
from jax.experimental.pallas import tpu_sc as plsc

B = 256
NQ = B // 128


def _tc_taps(feats, wcat, wc, c_out, n_pad, bn):
    n, c_in = feats.shape
    kkc = wcat.shape[1]

    def body(f_ref, w_ref, wc_ref, y_ref, yc_ref):
        f = f_ref[...]
        y = jnp.dot(f, w_ref[...], preferred_element_type=jnp.float32)
        y_ref[...] = y.astype(jnp.bfloat16)
        yc_ref[...] = jnp.dot(f, wc_ref[...],
                              preferred_element_type=jnp.float32)

    return pl.pallas_call(
        body,
        grid=(pl.cdiv(n_pad, bn),),
        in_specs=[
            pl.BlockSpec((bn, c_in), lambda i: (i, 0)),
            pl.BlockSpec((c_in, kkc), lambda i: (0, 0)),
            pl.BlockSpec((c_in, c_out), lambda i: (0, 0)),
        ],
        out_specs=[
            pl.BlockSpec((bn, kkc), lambda i: (i, 0)),
            pl.BlockSpec((bn, c_out), lambda i: (i, 0)),
        ],
        out_shape=[
            jax.ShapeDtypeStruct((n, kkc), jnp.bfloat16),
            jax.ShapeDtypeStruct((n_pad, c_out), jnp.float32),
        ],
    )(feats, wcat, wc)


def _sc_scatter(y2, yc, imap_p, omap_p, kpos_p, n_pad, c_out, kk, nseg):
    mesh = plsc.VectorSubcoreMesh(core_axis_name="c", subcore_axis_name="s",
                                  num_cores=2, num_subcores=16)
    nw = 32
    rw = n_pad // nw
    h = n_pad // 2

    center = nseg // 2

    def body(y2_h, yc_h, imap_h, omap_h, kpos_h, out_h,
             kpos_v, b_v, bs_idx, bs_val, raw_i0, raw_o0, raw_i1, raw_o1,
             idx_i0, idx_o0, idx_i1, idx_o1, gw0, gw1, gf0, gf1, acc,
             sem, sem_i, sem_g, sem_s0, sem_s1):
        bufs = ((raw_i0, raw_o0, idx_i0, idx_o0, gw0, gf0, sem_s0),
                (raw_i1, raw_o1, idx_i1, idx_o1, gw1, gf1, sem_s1))
        c = lax.axis_index("c")
        s = lax.axis_index("s")
        w = c * 16 + s
        base_row = w * rw
        rowbase = c * h
        pltpu.sync_copy(kpos_h, kpos_v)
        pltpu.sync_copy(yc_h.at[pl.ds(base_row, rw)],
                        acc.at[pl.ds(s * rw, rw)])
        lanes = lax.iota(jnp.int32, 16)

        k0 = kpos_v[pl.ds(0, 16)]
        k1 = kpos_v[pl.ds(1, 16)]
        k2 = kpos_v[pl.ds(16, 16)]
        k3 = kpos_v[pl.ds(17, 16)]
        k_m = k2[nseg - 16]
        in2 = lax.shift_right_logical(lanes - (nseg - 16), 31)
        k2a = k2 * in2 + k_m * (1 - in2)
        k3a = k3 * in2 + k_m * (1 - in2)
        los = [k0, k2a, k0, k2a]
        his = [k1, k3a, k1, k3a]
        for _ in range(17):
            mids = []
            for q in range(4):
                mid = lax.div(los[q] + his[q], 2)
                mids.append(mid)
                bs_idx[pl.ds(q * 16, 16)] = mid
            pltpu.async_copy(omap_h.at[bs_idx], bs_val, sem).wait()
            for q in range(4):
                ai = lax.shift_right_logical(los[q] - his[q], 31)
                target = (w + q // 2) * rw
                v = bs_val[pl.ds(q * 16, 16)]
                ci = lax.shift_right_logical(v - target, 31)
                go = ai * ci
                los[q] = los[q] + go * (mids[q] + 1 - los[q])
                stay = ai * (1 - ci)
                his[q] = his[q] + stay * (mids[q] - his[q])
        for q in range(4):
            b_v[pl.ds(q * 16, 16)] = los[q]

        def _scalar_at(i):
            v = b_v[pl.ds(i, 16)]
            return v[0]

        def seg_body(j, carry):
            start = _scalar_at(j)
            end = _scalar_at(32 + j)
            k_of = j + lax.div(j + center, nseg)
            a = start - lax.rem(start, 8)
            nblocks = lax.div(end - a + (B - 1), B)

            def _fire_idx(t, ri, ro):
                p = pl.multiple_of(a + t * B, 8)
                pltpu.async_copy(imap_h.at[pl.ds(p, B)], ri, sem_i)
                pltpu.async_copy(omap_h.at[pl.ds(p, B)], ro, sem_i)

            @pl.when(nblocks > 0)
            def _():
                _fire_idx(0, raw_i0, raw_o0)

            def outer(tt, carry2):
                for par in range(2):
                    t = 2 * tt + par
                    raw_i, raw_o, idx_i, idx_o, gw, gf, sem_s = bufs[par]
                    nraw_i, nraw_o = bufs[1 - par][0], bufs[1 - par][1]

                    @pl.when(t < nblocks)
                    def _():
                        p = pl.multiple_of(a + t * B, 8)
                        pltpu.make_async_copy(
                            imap_h.at[pl.ds(p, B)], raw_i, sem_i).wait()
                        pltpu.make_async_copy(
                            omap_h.at[pl.ds(p, B)], raw_o, sem_i).wait()

                        @pl.when(t + 1 < nblocks)
                        def _():
                            _fire_idx(t + 1, nraw_i, nraw_o)

                        @pl.when(t >= 2)
                        def _():
                            for q in range(NQ):
                                pltpu.make_async_copy(
                                    yc_h.at[pl.ds(0, 128)],
                                    gf.at[pl.ds(q * 128, 128)],
                                    sem_s).wait()
                        for u in range(B // 16):
                            posv = p + u * 16 + lanes
                            msk = (posv >= start) & (posv < end)
                            iv = raw_i[pl.ds(u * 16, 16)]
                            idx_i[u // 8, pl.ds((u % 8) * 16, 16)] = (
                                iv * kk + k_of)
                            ov = raw_o[pl.ds(u * 16, 16)]
                            idx_o[u // 8, pl.ds((u % 8) * 16, 16)] = (
                                jnp.where(msk, ov - rowbase, h))
                        gds = [pltpu.async_copy(
                                   y2_h.at[idx_i.at[q]],
                                   gw.at[pl.ds(q * 128, 128)], sem_g)
                               for q in range(NQ)]
                        for d in gds:
                            d.wait()
                        for r in range(B):
                            wv = gw[r, :]
                            gf[r, pl.ds(0, 16)] = lax.bitcast_convert_type(
                                lax.shift_left(wv, 16), jnp.float32)
                            gf[r, pl.ds(16, 16)] = lax.bitcast_convert_type(
                                wv & jnp.int32(-65536), jnp.float32)
                        for q in range(NQ):
                            pltpu.async_copy(gf.at[pl.ds(q * 128, 128)],
                                             acc.at[idx_o.at[q]], sem_s,
                                             add=True)
                return carry2

            lax.fori_loop(0, lax.div(nblocks + 1, 2), outer, carry)
            for par in range(2):
                gf, sem_s = bufs[par][5], bufs[par][6]

                @pl.when(nblocks > par)
                def _():
                    for q in range(NQ):
                        pltpu.make_async_copy(
                            yc_h.at[pl.ds(0, 128)],
                            gf.at[pl.ds(q * 128, 128)], sem_s).wait()
            return carry

        lax.fori_loop(0, nseg, seg_body, 0)
        pltpu.sync_copy(acc.at[pl.ds(s * rw, rw)],
                        out_h.at[pl.ds(base_row, rw)])

    return pl.kernel(
        body,
        out_type=jax.ShapeDtypeStruct((n_pad, c_out), jnp.float32),
        mesh=mesh,
        compiler_params=pltpu.CompilerParams(use_tc_tiling_on_sc=False),
        scratch_types=[
            pltpu.VMEM((48,), jnp.int32),
            pltpu.VMEM((96,), jnp.int32),
            pltpu.VMEM((64,), jnp.int32),
            pltpu.VMEM((64,), jnp.int32),
            pltpu.VMEM((B,), jnp.int32),
            pltpu.VMEM((B,), jnp.int32),
            pltpu.VMEM((B,), jnp.int32),
            pltpu.VMEM((B,), jnp.int32),
            pltpu.VMEM((NQ, 128), jnp.int32),
            pltpu.VMEM((NQ, 128), jnp.int32),
            pltpu.VMEM((NQ, 128), jnp.int32),
            pltpu.VMEM((NQ, 128), jnp.int32),
            pltpu.VMEM((B, c_out // 2), jnp.int32),
            pltpu.VMEM((B, c_out // 2), jnp.int32),
            pltpu.VMEM((B, c_out), jnp.float32),
            pltpu.VMEM((B, c_out), jnp.float32),
            pltpu.VMEM_SHARED((h + 8, c_out), jnp.float32),
            pltpu.SemaphoreType.DMA,
            pltpu.SemaphoreType.DMA,
            pltpu.SemaphoreType.DMA,
            pltpu.SemaphoreType.DMA,
            pltpu.SemaphoreType.DMA,
        ],
    )(y2, yc, imap_p, omap_p, kpos_p)


def kernel(feats, kernel, imap, omap, kpos):
    n, c_in = feats.shape
    kk, _, c_out = kernel.shape
    center = (kk - 1) // 2
    nseg = kk - 1
    bn = 2048
    assert n % 32 == 0, "row partition requires N divisible by 32"
    n_pad = n

    wcat = kernel.transpose(1, 0, 2).reshape(c_in, kk * c_out)
    ar = jnp.arange(c_out // 2, dtype=jnp.int32)
    order = jnp.stack([ar, ar + c_out // 2], axis=1).reshape(-1)
    colidx = (jnp.arange(kk, dtype=jnp.int32)[:, None] * c_out
              + order[None, :]).reshape(-1)
    wc_c = wcat[:, center * c_out:(center + 1) * c_out]
    y_bf, yc = _tc_taps(feats, wcat[:, colidx], wc_c, c_out, n_pad, bn)
    y2 = jax.lax.bitcast_convert_type(
        y_bf.reshape(n * kk, c_out // 2, 2), jnp.int32)

    imap_p = jnp.pad(imap, (0, B + 16))
    omap_p = jnp.pad(omap, (0, B + 16))
    kpos_p = jnp.pad(kpos, (0, 48 - kk))

    return _sc_scatter(y2, yc, imap_p, omap_p, kpos_p, n_pad, c_out, kk, nseg)


# speedup vs baseline: 80.8430x; 80.8430x over previous
import jax
import jax.numpy as jnp
from jax import lax
from jax.experimental import pallas as pl
from jax.experimental.pallas import tpu as pltpu
from jax.experimental.pallas import tpu_sc as plsc

B = 256
NQ = B // 128


def _tc_taps(feats, wcat, wc, c_out, n_pad, bn):
    n, c_in = feats.shape
    kkc2 = wcat[0].shape[1]

    def body(f_ref, wlo_ref, whi_ref, wc_ref, y_ref, yc_ref):
        f = f_ref[...]
        ylo = jnp.dot(f, wlo_ref[...], preferred_element_type=jnp.float32)
        yhi = jnp.dot(f, whi_ref[...], preferred_element_type=jnp.float32)
        lo = lax.bitcast_convert_type(ylo.astype(jnp.bfloat16),
                                      jnp.uint16).astype(jnp.int32)
        hi = lax.bitcast_convert_type(yhi.astype(jnp.bfloat16),
                                      jnp.uint16).astype(jnp.int32)
        y_ref[...] = lo | lax.shift_left(hi, 16)
        yc_ref[...] = jnp.dot(f, wc_ref[...],
                              preferred_element_type=jnp.float32)

    return pl.pallas_call(
        body,
        grid=(pl.cdiv(n_pad, bn),),
        in_specs=[
            pl.BlockSpec((bn, c_in), lambda i: (i, 0)),
            pl.BlockSpec((c_in, kkc2), lambda i: (0, 0)),
            pl.BlockSpec((c_in, kkc2), lambda i: (0, 0)),
            pl.BlockSpec((c_in, c_out), lambda i: (0, 0)),
        ],
        out_specs=[
            pl.BlockSpec((bn, kkc2), lambda i: (i, 0)),
            pl.BlockSpec((bn, c_out), lambda i: (i, 0)),
        ],
        out_shape=[
            jax.ShapeDtypeStruct((n, kkc2), jnp.int32),
            jax.ShapeDtypeStruct((n_pad, c_out), jnp.float32),
        ],
    )(feats, wcat[0], wcat[1], wc)


def _sc_scatter(y2, yc, imap_p, omap_p, kpos_p, n_pad, c_out, kk, nseg):
    mesh = plsc.VectorSubcoreMesh(core_axis_name="c", subcore_axis_name="s",
                                  num_cores=2, num_subcores=16)
    nw = 32
    rw = n_pad // nw
    h = n_pad // 2

    center = nseg // 2

    def body(y2_h, yc_h, imap_h, omap_h, kpos_h, out_h,
             kpos_v, b_v, bs_idx, bs_val, raw_i0, raw_o0, raw_i1, raw_o1,
             idx_i0, idx_o0, idx_i1, idx_o1, gw0, gw1, gf0, gf1, acc,
             sem, sem_i, sem_g, sem_s0, sem_s1):
        bufs = ((raw_i0, raw_o0, idx_i0, idx_o0, gw0, gf0, sem_s0),
                (raw_i1, raw_o1, idx_i1, idx_o1, gw1, gf1, sem_s1))
        c = lax.axis_index("c")
        s = lax.axis_index("s")
        w = c * 16 + s
        base_row = w * rw
        rowbase = c * h
        pltpu.sync_copy(kpos_h, kpos_v)
        pltpu.sync_copy(yc_h.at[pl.ds(base_row, rw)],
                        acc.at[pl.ds(s * rw, rw)])
        lanes = lax.iota(jnp.int32, 16)

        k0 = kpos_v[pl.ds(0, 16)]
        k1 = kpos_v[pl.ds(1, 16)]
        k2 = kpos_v[pl.ds(16, 16)]
        k3 = kpos_v[pl.ds(17, 16)]
        k_m = k2[nseg - 16]
        in2 = lax.shift_right_logical(lanes - (nseg - 16), 31)
        k2a = k2 * in2 + k_m * (1 - in2)
        k3a = k3 * in2 + k_m * (1 - in2)
        los = [k0, k2a, k0, k2a]
        his = [k1, k3a, k1, k3a]
        for _ in range(17):
            mids = []
            for q in range(4):
                mid = lax.div(los[q] + his[q], 2)
                mids.append(mid)
                bs_idx[pl.ds(q * 16, 16)] = mid
            pltpu.async_copy(omap_h.at[bs_idx], bs_val, sem).wait()
            for q in range(4):
                ai = lax.shift_right_logical(los[q] - his[q], 31)
                target = (w + q // 2) * rw
                v = bs_val[pl.ds(q * 16, 16)]
                ci = lax.shift_right_logical(v - target, 31)
                go = ai * ci
                los[q] = los[q] + go * (mids[q] + 1 - los[q])
                stay = ai * (1 - ci)
                his[q] = his[q] + stay * (mids[q] - his[q])
        for q in range(4):
            b_v[pl.ds(q * 16, 16)] = los[q]

        def _scalar_at(i):
            v = b_v[pl.ds(i, 16)]
            return v[0]

        def seg_body(j, carry):
            start = _scalar_at(j)
            end = _scalar_at(32 + j)
            k_of = j + lax.div(j + center, nseg)
            a = start - lax.rem(start, 8)
            nblocks = lax.div(end - a + (B - 1), B)

            def _fire_idx(t, ri, ro):
                p = pl.multiple_of(a + t * B, 8)
                pltpu.async_copy(imap_h.at[pl.ds(p, B)], ri, sem_i)
                pltpu.async_copy(omap_h.at[pl.ds(p, B)], ro, sem_i)

            @pl.when(nblocks > 0)
            def _():
                _fire_idx(0, raw_i0, raw_o0)

            def outer(tt, carry2):
                for par in range(2):
                    t = 2 * tt + par
                    raw_i, raw_o, idx_i, idx_o, gw, gf, sem_s = bufs[par]
                    nraw_i, nraw_o = bufs[1 - par][0], bufs[1 - par][1]

                    @pl.when(t < nblocks)
                    def _():
                        p = pl.multiple_of(a + t * B, 8)
                        pltpu.make_async_copy(
                            imap_h.at[pl.ds(p, B)], raw_i, sem_i).wait()
                        pltpu.make_async_copy(
                            omap_h.at[pl.ds(p, B)], raw_o, sem_i).wait()

                        @pl.when(t + 1 < nblocks)
                        def _():
                            _fire_idx(t + 1, nraw_i, nraw_o)

                        @pl.when(t >= 2)
                        def _():
                            for q in range(NQ):
                                pltpu.make_async_copy(
                                    yc_h.at[pl.ds(0, 128)],
                                    gf.at[pl.ds(q * 128, 128)],
                                    sem_s).wait()
                        for u in range(B // 16):
                            posv = p + u * 16 + lanes
                            msk = (posv >= start) & (posv < end)
                            iv = raw_i[pl.ds(u * 16, 16)]
                            idx_i[u // 8, pl.ds((u % 8) * 16, 16)] = (
                                iv * kk + k_of)
                            ov = raw_o[pl.ds(u * 16, 16)]
                            idx_o[u // 8, pl.ds((u % 8) * 16, 16)] = (
                                jnp.where(msk, ov - rowbase, h))
                        gds = [pltpu.async_copy(
                                   y2_h.at[idx_i.at[q]],
                                   gw.at[pl.ds(q * 128, 128)], sem_g)
                               for q in range(NQ)]
                        for d in gds:
                            d.wait()
                        for r in range(B):
                            wv = gw[r, :]
                            gf[r, pl.ds(0, 16)] = lax.bitcast_convert_type(
                                lax.shift_left(wv, 16), jnp.float32)
                            gf[r, pl.ds(16, 16)] = lax.bitcast_convert_type(
                                wv & jnp.int32(-65536), jnp.float32)
                        for q in range(NQ):
                            pltpu.async_copy(gf.at[pl.ds(q * 128, 128)],
                                             acc.at[idx_o.at[q]], sem_s,
                                             add=True)
                return carry2

            lax.fori_loop(0, lax.div(nblocks + 1, 2), outer, carry)
            for par in range(2):
                gf, sem_s = bufs[par][5], bufs[par][6]

                @pl.when(nblocks > par)
                def _():
                    for q in range(NQ):
                        pltpu.make_async_copy(
                            yc_h.at[pl.ds(0, 128)],
                            gf.at[pl.ds(q * 128, 128)], sem_s).wait()
            return carry

        lax.fori_loop(0, nseg, seg_body, 0)
        pltpu.sync_copy(acc.at[pl.ds(s * rw, rw)],
                        out_h.at[pl.ds(base_row, rw)])

    return pl.kernel(
        body,
        out_type=jax.ShapeDtypeStruct((n_pad, c_out), jnp.float32),
        mesh=mesh,
        compiler_params=pltpu.CompilerParams(use_tc_tiling_on_sc=False),
        scratch_types=[
            pltpu.VMEM((48,), jnp.int32),
            pltpu.VMEM((96,), jnp.int32),
            pltpu.VMEM((64,), jnp.int32),
            pltpu.VMEM((64,), jnp.int32),
            pltpu.VMEM((B,), jnp.int32),
            pltpu.VMEM((B,), jnp.int32),
            pltpu.VMEM((B,), jnp.int32),
            pltpu.VMEM((B,), jnp.int32),
            pltpu.VMEM((NQ, 128), jnp.int32),
            pltpu.VMEM((NQ, 128), jnp.int32),
            pltpu.VMEM((NQ, 128), jnp.int32),
            pltpu.VMEM((NQ, 128), jnp.int32),
            pltpu.VMEM((B, c_out // 2), jnp.int32),
            pltpu.VMEM((B, c_out // 2), jnp.int32),
            pltpu.VMEM((B, c_out), jnp.float32),
            pltpu.VMEM((B, c_out), jnp.float32),
            pltpu.VMEM_SHARED((h + 8, c_out), jnp.float32),
            pltpu.SemaphoreType.DMA,
            pltpu.SemaphoreType.DMA,
            pltpu.SemaphoreType.DMA,
            pltpu.SemaphoreType.DMA,
            pltpu.SemaphoreType.DMA,
        ],
    )(y2, yc, imap_p, omap_p, kpos_p)


def kernel(feats, kernel, imap, omap, kpos):
    n, c_in = feats.shape
    kk, _, c_out = kernel.shape
    center = (kk - 1) // 2
    nseg = kk - 1
    bn = 2048
    assert n % 32 == 0, "row partition requires N divisible by 32"
    n_pad = n

    wfull = kernel.transpose(1, 0, 2)
    wc_c = wfull[:, center, :]
    wlo = wfull[:, :, :c_out // 2].reshape(c_in, kk * c_out // 2)
    whi = wfull[:, :, c_out // 2:].reshape(c_in, kk * c_out // 2)
    y, yc = _tc_taps(feats, (wlo, whi), wc_c, c_out, n_pad, bn)
    y2 = y.reshape(n * kk, c_out // 2)

    imap_p = jnp.pad(imap, (0, B + 16))
    omap_p = jnp.pad(omap, (0, B + 16))
    kpos_p = jnp.pad(kpos, (0, 48 - kk))

    return _sc_scatter(y2, yc, imap_p, omap_p, kpos_p, n_pad, c_out, kk, nseg)


# speedup vs baseline: 86.1995x; 1.0663x over previous
import jax
import jax.numpy as jnp
from jax import lax
from jax.experimental import pallas as pl
from jax.experimental.pallas import tpu as pltpu
from jax.experimental.pallas import tpu_sc as plsc

B = 256
NQ = B // 128


def _tc_taps(feats, wcat, wc, c_out, n_pad, bn):
    n, c_in = feats.shape
    kkc2 = wcat[0].shape[1]

    def body(f_ref, wlo_ref, whi_ref, wc_ref, y_ref, yc_ref):
        f = f_ref[...]
        ylo = jnp.dot(f, wlo_ref[...], preferred_element_type=jnp.float32)
        yhi = jnp.dot(f, whi_ref[...], preferred_element_type=jnp.float32)
        lo = lax.bitcast_convert_type(ylo.astype(jnp.bfloat16),
                                      jnp.uint16).astype(jnp.int32)
        hi = lax.bitcast_convert_type(yhi.astype(jnp.bfloat16),
                                      jnp.uint16).astype(jnp.int32)
        y_ref[...] = lo | lax.shift_left(hi, 16)
        yc_ref[...] = jnp.dot(f, wc_ref[...],
                              preferred_element_type=jnp.float32)

    return pl.pallas_call(
        body,
        grid=(pl.cdiv(n_pad, bn),),
        in_specs=[
            pl.BlockSpec((bn, c_in), lambda i: (i, 0)),
            pl.BlockSpec((c_in, kkc2), lambda i: (0, 0)),
            pl.BlockSpec((c_in, kkc2), lambda i: (0, 0)),
            pl.BlockSpec((c_in, c_out), lambda i: (0, 0)),
        ],
        out_specs=[
            pl.BlockSpec((bn, kkc2), lambda i: (i, 0)),
            pl.BlockSpec((bn, c_out), lambda i: (i, 0)),
        ],
        out_shape=[
            jax.ShapeDtypeStruct((n, kkc2), jnp.int32),
            jax.ShapeDtypeStruct((n_pad, c_out), jnp.float32),
        ],
    )(feats, wcat[0], wcat[1], wc)


def _sc_scatter(y2, yc, imap_p, omap_p, kpos_p, n_pad, c_out, kk, nseg):
    mesh = plsc.VectorSubcoreMesh(core_axis_name="c", subcore_axis_name="s",
                                  num_cores=2, num_subcores=16)
    nw = 32
    rw = n_pad // nw
    h = n_pad // 2

    center = nseg // 2

    def body(y2_h, yc_h, imap_h, omap_h, kpos_h, out_h,
             kpos_v, b_v, bs_idx, bs_val, raw_i0, raw_o0, raw_i1, raw_o1,
             idx_i0, idx_o0, idx_i1, idx_o1, gw0, gw1, gf0, gf1, acc,
             sem, sem_i, sem_g0, sem_g1, sem_s0, sem_s1):
        bufs = ((raw_i0, raw_o0, idx_i0, idx_o0, gw0, gf0, sem_g0, sem_s0),
                (raw_i1, raw_o1, idx_i1, idx_o1, gw1, gf1, sem_g1, sem_s1))
        c = lax.axis_index("c")
        s = lax.axis_index("s")
        w = c * 16 + s
        base_row = w * rw
        rowbase = c * h
        pltpu.sync_copy(kpos_h, kpos_v)
        pltpu.sync_copy(yc_h.at[pl.ds(base_row, rw)],
                        acc.at[pl.ds(s * rw, rw)])
        lanes = lax.iota(jnp.int32, 16)

        k0 = kpos_v[pl.ds(0, 16)]
        k1 = kpos_v[pl.ds(1, 16)]
        k2 = kpos_v[pl.ds(16, 16)]
        k3 = kpos_v[pl.ds(17, 16)]
        k_m = k2[nseg - 16]
        in2 = lax.shift_right_logical(lanes - (nseg - 16), 31)
        k2a = k2 * in2 + k_m * (1 - in2)
        k3a = k3 * in2 + k_m * (1 - in2)
        los = [k0, k2a, k0, k2a]
        his = [k1, k3a, k1, k3a]
        for _ in range(17):
            mids = []
            for q in range(4):
                mid = lax.div(los[q] + his[q], 2)
                mids.append(mid)
                bs_idx[pl.ds(q * 16, 16)] = mid
            pltpu.async_copy(omap_h.at[bs_idx], bs_val, sem).wait()
            for q in range(4):
                ai = lax.shift_right_logical(los[q] - his[q], 31)
                target = (w + q // 2) * rw
                v = bs_val[pl.ds(q * 16, 16)]
                ci = lax.shift_right_logical(v - target, 31)
                go = ai * ci
                los[q] = los[q] + go * (mids[q] + 1 - los[q])
                stay = ai * (1 - ci)
                his[q] = his[q] + stay * (mids[q] - his[q])
        for q in range(4):
            b_v[pl.ds(q * 16, 16)] = los[q]

        def _scalar_at(i):
            v = b_v[pl.ds(i, 16)]
            return v[0]

        def seg_body(j, carry):
            start = _scalar_at(j)
            end = _scalar_at(32 + j)
            k_of = j + lax.div(j + center, nseg)
            a = start - lax.rem(start, 8)
            nblocks = lax.div(end - a + (B - 1), B)

            def _fire_idx(t, ri, ro):
                p = pl.multiple_of(a + t * B, 8)
                pltpu.async_copy(imap_h.at[pl.ds(p, B)], ri, sem_i)
                pltpu.async_copy(omap_h.at[pl.ds(p, B)], ro, sem_i)

            @pl.when(nblocks > 0)
            def _():
                _fire_idx(0, raw_i0, raw_o0)

            def outer(tt, carry2):
                for par in range(2):
                    t = 2 * tt + par
                    (raw_i, raw_o, idx_i, idx_o, gw, gf, sem_g,
                     sem_s) = bufs[par]
                    (praw_i, praw_o, pidx_i, pidx_o, pgw, pgf, psem_g,
                     psem_s) = bufs[1 - par]

                    @pl.when((t >= 2) & (t - 2 < nblocks))
                    def _():
                        for q in range(NQ):
                            pltpu.make_async_copy(
                                yc_h.at[pl.ds(0, 128)],
                                gf.at[pl.ds(q * 128, 128)],
                                sem_s).wait()

                    @pl.when(t < nblocks)
                    def _():
                        p = pl.multiple_of(a + t * B, 8)
                        pltpu.make_async_copy(
                            imap_h.at[pl.ds(p, B)], raw_i, sem_i).wait()
                        pltpu.make_async_copy(
                            omap_h.at[pl.ds(p, B)], raw_o, sem_i).wait()

                        @pl.when(t + 1 < nblocks)
                        def _():
                            _fire_idx(t + 1, praw_i, praw_o)
                        for u in range(B // 16):
                            posv = p + u * 16 + lanes
                            msk = (posv >= start) & (posv < end)
                            iv = raw_i[pl.ds(u * 16, 16)]
                            idx_i[u // 8, pl.ds((u % 8) * 16, 16)] = (
                                iv * kk + k_of)
                            ov = raw_o[pl.ds(u * 16, 16)]
                            idx_o[u // 8, pl.ds((u % 8) * 16, 16)] = (
                                jnp.where(msk, ov - rowbase, h))
                        for q in range(NQ):
                            pltpu.async_copy(
                                y2_h.at[idx_i.at[q]],
                                gw.at[pl.ds(q * 128, 128)], sem_g)

                    @pl.when((t >= 1) & (t - 1 < nblocks))
                    def _():
                        for q in range(NQ):
                            pltpu.make_async_copy(
                                y2_h.at[pl.ds(0, 128)],
                                pgw.at[pl.ds(q * 128, 128)], psem_g).wait()
                        for r in range(B):
                            wv = pgw[r, :]
                            pgf[r, pl.ds(0, 16)] = lax.bitcast_convert_type(
                                lax.shift_left(wv, 16), jnp.float32)
                            pgf[r, pl.ds(16, 16)] = lax.bitcast_convert_type(
                                wv & jnp.int32(-65536), jnp.float32)
                        for q in range(NQ):
                            pltpu.async_copy(pgf.at[pl.ds(q * 128, 128)],
                                             acc.at[pidx_o.at[q]], psem_s,
                                             add=True)
                return carry2

            lax.fori_loop(0, lax.div(nblocks + 2, 2) + 1, outer, carry)
            return carry

        lax.fori_loop(0, nseg, seg_body, 0)
        pltpu.sync_copy(acc.at[pl.ds(s * rw, rw)],
                        out_h.at[pl.ds(base_row, rw)])

    return pl.kernel(
        body,
        out_type=jax.ShapeDtypeStruct((n_pad, c_out), jnp.float32),
        mesh=mesh,
        compiler_params=pltpu.CompilerParams(use_tc_tiling_on_sc=False),
        scratch_types=[
            pltpu.VMEM((48,), jnp.int32),
            pltpu.VMEM((96,), jnp.int32),
            pltpu.VMEM((64,), jnp.int32),
            pltpu.VMEM((64,), jnp.int32),
            pltpu.VMEM((B,), jnp.int32),
            pltpu.VMEM((B,), jnp.int32),
            pltpu.VMEM((B,), jnp.int32),
            pltpu.VMEM((B,), jnp.int32),
            pltpu.VMEM((NQ, 128), jnp.int32),
            pltpu.VMEM((NQ, 128), jnp.int32),
            pltpu.VMEM((NQ, 128), jnp.int32),
            pltpu.VMEM((NQ, 128), jnp.int32),
            pltpu.VMEM((B, c_out // 2), jnp.int32),
            pltpu.VMEM((B, c_out // 2), jnp.int32),
            pltpu.VMEM((B, c_out), jnp.float32),
            pltpu.VMEM((B, c_out), jnp.float32),
            pltpu.VMEM_SHARED((h + 8, c_out), jnp.float32),
            pltpu.SemaphoreType.DMA,
            pltpu.SemaphoreType.DMA,
            pltpu.SemaphoreType.DMA,
            pltpu.SemaphoreType.DMA,
            pltpu.SemaphoreType.DMA,
            pltpu.SemaphoreType.DMA,
        ],
    )(y2, yc, imap_p, omap_p, kpos_p)


def kernel(feats, kernel, imap, omap, kpos):
    n, c_in = feats.shape
    kk, _, c_out = kernel.shape
    center = (kk - 1) // 2
    nseg = kk - 1
    bn = 2048
    assert n % 32 == 0, "row partition requires N divisible by 32"
    n_pad = n

    wfull = kernel.transpose(1, 0, 2)
    wc_c = wfull[:, center, :]
    wlo = wfull[:, :, :c_out // 2].reshape(c_in, kk * c_out // 2)
    whi = wfull[:, :, c_out // 2:].reshape(c_in, kk * c_out // 2)
    y, yc = _tc_taps(feats, (wlo, whi), wc_c, c_out, n_pad, bn)
    y2 = y.reshape(n * kk, c_out // 2)

    imap_p = jnp.pad(imap, (0, B + 16))
    omap_p = jnp.pad(omap, (0, B + 16))
    kpos_p = jnp.pad(kpos, (0, 48 - kk))

    return _sc_scatter(y2, yc, imap_p, omap_p, kpos_p, n_pad, c_out, kk, nseg)
